# baseline (speedup 1.0000x reference)
import functools

import jax
import jax.numpy as jnp
from jax import lax
from jax.experimental import pallas as pl
from jax.experimental.pallas import tpu as pltpu
from jax.experimental.pallas import tpu_sc as plsc

_H, _W, _C, _N = 2048, 2048, 8, 1048576
_NC, _NS, _L = 2, 16, 16
_NW = _NC * _NS
_NPW = _N // _NW
_B = 1024
_NCHUNK = _NPW // _B
_JR = _B // 128

_f32 = jnp.float32
_i32 = jnp.int32


def _vperm(x, idx):
    dnums = lax.GatherDimensionNumbers(
        offset_dims=(), collapsed_slice_dims=(0,), start_index_map=(0,))
    return lax.gather(x, idx[:, None], dnums, (1,),
                      mode=lax.GatherScatterMode.PROMISE_IN_BOUNDS)


def _sc_body(tex_hbm, uv_hbm, vals_hbm, hom_hbm, vnn_hbm,
             uv_v, i00, i01, i10, i11, wxr, wyr,
             t00, t01, t10, t11, vnn_v, vals_v, hom_v, sem):
    wid = lax.axis_index("s") * _NC + lax.axis_index("c")

    iota = lax.iota(_i32, _L)
    iota2 = iota * 2
    colc = iota & 7
    halfc = iota >> 3
    hsel = 7 + halfc * 8
    sidxc = colc + halfc * 7
    mask_v = colc != 7
    mask_h = colc == 7

    def chunk_body(ci, carry):
        base = wid * _NPW + ci * _B
        pltpu.sync_copy(uv_hbm.at[pl.ds(base * 2, 2 * _B)], uv_v)

        def p1(i, c):
            j = i // 8
            cc = (i % 8) * 16
            o2 = iota2 + i * 32
            u = plsc.load_gather(uv_v, [o2])
            v = plsc.load_gather(uv_v, [o2 + 1])
            x = u * float(_W - 1)
            y = v * float(_H - 1)
            xi = jnp.minimum(x.astype(_i32), _W - 2)
            yi = jnp.minimum(y.astype(_i32), _H - 2)
            wx = x - xi.astype(_f32)
            wy = y - yi.astype(_f32)
            k00 = (yi << 11) + xi
            i00[j, pl.ds(cc, 16)] = k00
            i01[j, pl.ds(cc, 16)] = k00 + 1
            i10[j, pl.ds(cc, 16)] = k00 + _W
            i11[j, pl.ds(cc, 16)] = k00 + _W + 1
            wxr[pl.ds(i * 16, 16)] = wx
            wyr[pl.ds(i * 16, 16)] = wy
            return c
        lax.fori_loop(0, _B // 16, p1, 0)

        copies = []
        for j in range(_JR):
            d = pl.ds(j * 128, 128)
            copies.append(pltpu.async_copy(tex_hbm.at[i00.at[j]], t00.at[d, :], sem))
            copies.append(pltpu.async_copy(tex_hbm.at[i01.at[j]], t01.at[d, :], sem))
            copies.append(pltpu.async_copy(tex_hbm.at[i10.at[j]], t10.at[d, :], sem))
            copies.append(pltpu.async_copy(tex_hbm.at[i11.at[j]], t11.at[d, :], sem))
        for c in copies:
            c.wait()

        def p2(i, c):
            pidx = halfc + i * 2
            wxb = plsc.load_gather(wxr, [pidx])
            wyb = plsc.load_gather(wyr, [pidx])
            cx = 1.0 - wxb
            cy = 1.0 - wyb
            a = plsc.load_gather(t00, [pidx, colc])
            b = plsc.load_gather(t01, [pidx, colc])
            g = plsc.load_gather(t10, [pidx, colc])
            d = plsc.load_gather(t11, [pidx, colc])
            vnn16 = a * cx * cy + b * wxb * cy + g * cx * wyb + d * wxb * wyb
            vnn_v[pl.ds(i * 16, 16)] = vnn16
            hb = _vperm(vnn16, hsel)
            vals16 = vnn16 / (hb + 1e-05)
            plsc.store_scatter(vals_v, [sidxc + i * 14], vals16, mask=mask_v)
            plsc.store_scatter(hom_v, [pidx], vnn16, mask=mask_h)
            return c
        lax.fori_loop(0, _B // 2, p2, 0)

        pltpu.sync_copy(vals_v, vals_hbm.at[pl.ds(base * 7, 7 * _B)])
        pltpu.sync_copy(hom_v, hom_hbm.at[pl.ds(base, _B)])
        pltpu.sync_copy(vnn_v, vnn_hbm.at[pl.ds(base * 8, 8 * _B)])
        return carry

    lax.fori_loop(0, _NCHUNK, chunk_body, 0)


_sc_kernel = functools.partial(
    pl.kernel,
    out_type=(
        jax.ShapeDtypeStruct((_N * 7,), _f32),
        jax.ShapeDtypeStruct((_N,), _f32),
        jax.ShapeDtypeStruct((_N * 8,), _f32),
    ),
    mesh=plsc.VectorSubcoreMesh(core_axis_name="c", subcore_axis_name="s"),
    compiler_params=pltpu.CompilerParams(
        needs_layout_passes=False, use_tc_tiling_on_sc=False),
    scratch_types=[
        pltpu.VMEM((2 * _B,), _f32),
        pltpu.VMEM((_JR, 128), _i32),
        pltpu.VMEM((_JR, 128), _i32),
        pltpu.VMEM((_JR, 128), _i32),
        pltpu.VMEM((_JR, 128), _i32),
        pltpu.VMEM((_B,), _f32),
        pltpu.VMEM((_B,), _f32),
        pltpu.VMEM((_B, _C), _f32),
        pltpu.VMEM((_B, _C), _f32),
        pltpu.VMEM((_B, _C), _f32),
        pltpu.VMEM((_B, _C), _f32),
        pltpu.VMEM((_B * _C,), _f32),
        pltpu.VMEM((_B * 7,), _f32),
        pltpu.VMEM((_B,), _f32),
        pltpu.SemaphoreType.DMA,
    ],
)(_sc_body)


def kernel(texture, uv_tensor):
    tex = texture.reshape(_H * _W, _C)
    uv = uv_tensor.reshape(-1)
    vals, hom, vnn = _sc_kernel(tex, uv)
    return (vals.reshape(_N, 7), hom.reshape(_N, 1), vnn.reshape(_N, _C))
